# baseline (speedup 1.0000x reference)
import functools

import jax
import jax.numpy as jnp
from jax import lax
from jax.experimental import pallas as pl
from jax.experimental.pallas import tpu as pltpu
from jax.experimental.pallas import tpu_sc as plsc

N = 1048576
VOL = 256
VOL3 = VOL * VOL * VOL
NW = 32
SPW = N // NW
C = 2048
G = C // 16
NCHUNK = SPW // C


def _tec_body(smp_hbm, table_hbm, out_hbm,
              smp_v, idx_v, val_v, wx_v, wy_v, wz_v, out_v, sem):
    wid = lax.axis_index("s") * 2 + lax.axis_index("c")

    def chunk_body(k, carry):
        base = wid * SPW + k * C

        for j in range(7):
            pltpu.sync_copy(smp_hbm.at[pl.ds(j * N + base, C)],
                            smp_v.at[pl.ds(j * C, C)])

        def idx_body(g, carry2):
            r = g * 16
            c0 = smp_v[pl.ds(0 * C + r, 16)]
            c1 = smp_v[pl.ds(1 * C + r, 16)]
            c2 = smp_v[pl.ds(2 * C + r, 16)]
            c3 = smp_v[pl.ds(3 * C + r, 16)]
            c4 = smp_v[pl.ds(4 * C + r, 16)]
            c5 = smp_v[pl.ds(5 * C + r, 16)]
            fl = smp_v[pl.ds(6 * C + r, 16)]
            is_yin = fl == 0.0
            x = (jnp.where(is_yin, c0, c3) + 1.0) * 127.5
            y = (jnp.where(is_yin, c1, c4) + 1.0) * 127.5
            z = (jnp.where(is_yin, c2, c5) + 1.0) * 127.5
            xi = x.astype(jnp.int32)
            yi = y.astype(jnp.int32)
            zi = z.astype(jnp.int32)
            wx_v[pl.ds(g * 16, 16)] = x - xi.astype(jnp.float32)
            wy_v[pl.ds(g * 16, 16)] = y - yi.astype(jnp.float32)
            wz_v[pl.ds(g * 16, 16)] = z - zi.astype(jnp.float32)
            vsel = jnp.where(is_yin, 0, VOL3)
            b000 = vsel + zi * (VOL * VOL) + yi * VOL + xi
            s = g * 16
            idx_v[pl.ds(0 * C + s, 16)] = b000
            idx_v[pl.ds(1 * C + s, 16)] = b000 + 1
            idx_v[pl.ds(2 * C + s, 16)] = b000 + VOL
            idx_v[pl.ds(3 * C + s, 16)] = b000 + (VOL + 1)
            idx_v[pl.ds(4 * C + s, 16)] = b000 + VOL * VOL
            idx_v[pl.ds(5 * C + s, 16)] = b000 + (VOL * VOL + 1)
            idx_v[pl.ds(6 * C + s, 16)] = b000 + (VOL * VOL + VOL)
            idx_v[pl.ds(7 * C + s, 16)] = b000 + (VOL * VOL + VOL + 1)
            return carry2

        lax.fori_loop(0, G, idx_body, 0, unroll=False)

        pltpu.async_copy(table_hbm.at[idx_v], val_v, sem).wait()

        def mix_body(g, carry2):
            s = g * 16
            wx = wx_v[pl.ds(s, 16)]
            wy = wy_v[pl.ds(s, 16)]
            wz = wz_v[pl.ds(s, 16)]
            v000 = val_v[pl.ds(0 * C + s, 16)]
            v001 = val_v[pl.ds(1 * C + s, 16)]
            v010 = val_v[pl.ds(2 * C + s, 16)]
            v011 = val_v[pl.ds(3 * C + s, 16)]
            v100 = val_v[pl.ds(4 * C + s, 16)]
            v101 = val_v[pl.ds(5 * C + s, 16)]
            v110 = val_v[pl.ds(6 * C + s, 16)]
            v111 = val_v[pl.ds(7 * C + s, 16)]
            a00 = v000 + wx * (v001 - v000)
            a01 = v010 + wx * (v011 - v010)
            a10 = v100 + wx * (v101 - v100)
            a11 = v110 + wx * (v111 - v110)
            b0 = a00 + wy * (a01 - a00)
            b1 = a10 + wy * (a11 - a10)
            out_v[pl.ds(s, 16)] = b0 + wz * (b1 - b0)
            return carry2

        lax.fori_loop(0, G, mix_body, 0, unroll=False)

        pltpu.sync_copy(out_v, out_hbm.at[pl.ds(base, C)])
        return carry

    lax.fori_loop(0, NCHUNK, chunk_body, 0, unroll=False)


@functools.partial(jax.jit, static_argnames=())
def _run(flat_samples, table):
    mesh = plsc.VectorSubcoreMesh(core_axis_name="c", subcore_axis_name="s")
    f = functools.partial(
        pl.kernel,
        mesh=mesh,
        out_type=jax.ShapeDtypeStruct((N,), jnp.float32),
        scratch_types=[
            pltpu.VMEM((C * 7,), jnp.float32),
            pltpu.VMEM((8 * C,), jnp.int32),
            pltpu.VMEM((8 * C,), jnp.float32),
            pltpu.VMEM((C,), jnp.float32),
            pltpu.VMEM((C,), jnp.float32),
            pltpu.VMEM((C,), jnp.float32),
            pltpu.VMEM((C,), jnp.float32),
            pltpu.SemaphoreType.DMA,
        ],
    )(_tec_body)
    return f(flat_samples, table)


def kernel(norm_samples, alpha_volume_yin, alpha_volume_yang):
    flat_samples = norm_samples.T.reshape(-1)
    table = jnp.concatenate(
        [alpha_volume_yin.reshape(-1), alpha_volume_yang.reshape(-1)])
    return _run(flat_samples, table)
